# baseline (speedup 1.0000x reference)
import functools

import jax
import jax.numpy as jnp
from jax import lax
from jax.experimental import pallas as pl
from jax.experimental.pallas import tpu as pltpu
from jax.experimental.pallas import tpu_sc as plsc

N = 10000
NP = 10240
E = 320000
D = 128
H = 64
K = 8

NC = 2
NS = 16
NW = NC * NS
EPW = E // NW
CH = 8000
NCH = E // CH
FPW = H // NW
SB = 14

_mesh = plsc.VectorSubcoreMesh(core_axis_name="c", subcore_axis_name="s")
_sc_params = pltpu.CompilerParams(needs_layout_passes=False)


def _wid():
    return lax.axis_index("s") * NC + lax.axis_index("c")


@functools.partial(
    pl.kernel,
    out_type=jax.ShapeDtypeStruct((NW * NP,), jnp.float32),
    mesh=_mesh,
    compiler_params=_sc_params,
    scratch_types=[
        pltpu.VMEM((EPW,), jnp.int32),
        pltpu.VMEM((NP,), jnp.float32),
    ],
)
def _deg_kernel(src_hbm, out_hbm, src_v, deg_v):
    wid = _wid()
    zero16 = jnp.zeros((16,), jnp.float32)

    @plsc.parallel_loop(0, NP, step=16, unroll=4)
    def _(i):
        deg_v[pl.ds(i, 16)] = zero16

    pltpu.sync_copy(src_hbm.at[pl.ds(wid * EPW, EPW)], src_v)
    one16 = jnp.ones((16,), jnp.float32)

    @plsc.parallel_loop(0, EPW, step=16, unroll=8)
    def _(i):
        s = src_v[pl.ds(i, 16)]
        plsc.addupdate_scatter(deg_v, [s], one16)

    pltpu.sync_copy(deg_v, out_hbm.at[pl.ds(wid * NP, NP)])


def _dense_body(x_ref, wf_ref, degp_ref, zt_ref, d2_ref, dsq_ref):
    deg = jnp.sum(degp_ref[...], axis=0, keepdims=True)
    dinv = jnp.where(deg > 0.0, lax.rsqrt(deg), 0.0)
    d2_ref[...] = dinv * dinv
    dsq_ref[...] = jnp.where(deg > 0.0, jnp.sqrt(deg), 0.0)
    zt = lax.dot_general(
        wf_ref[...], x_ref[...], (((1,), (1,)), ((), ())),
        preferred_element_type=jnp.float32)
    zt_ref[...] = zt * dinv


def _dense(xp, wf, degp):
    return pl.pallas_call(
        _dense_body,
        out_shape=(
            jax.ShapeDtypeStruct((K * H, NP), jnp.float32),
            jax.ShapeDtypeStruct((1, NP), jnp.float32),
            jax.ShapeDtypeStruct((1, NP), jnp.float32),
        ),
    )(xp, wf, degp)


@functools.partial(
    pl.kernel,
    out_type=jax.ShapeDtypeStruct((H * NP,), jnp.float32),
    mesh=_mesh,
    compiler_params=_sc_params,
    scratch_types=[
        pltpu.VMEM((2 * NP,), jnp.float32),
        pltpu.VMEM((2 * NP,), jnp.float32),
        pltpu.VMEM((2 * NP,), jnp.float32),
        pltpu.VMEM((2 * NP,), jnp.float32),
        pltpu.VMEM((2 * NP,), jnp.float32),
        pltpu.VMEM((NP,), jnp.float32),
        pltpu.VMEM((CH,), jnp.int32),
        pltpu.VMEM((CH,), jnp.int32),
        pltpu.SemaphoreType.DMA,
        pltpu.SemaphoreType.DMA,
        pltpu.SemaphoreType.DMA,
    ],
)
def _clenshaw_kernel(packed_hbm, zt_hbm, d2_hbm, st_hbm, P, Q, O, O2, Z, D2,
                     EB0, EB1, sem0, sem1, semz):
    wid = _wid()
    h0 = wid * FPW
    zero16 = jnp.zeros((16,), jnp.float32)
    nsplat = jnp.full((16,), NP, jnp.int32)
    msk = jnp.full((16,), (1 << SB) - 1, jnp.int32)

    @plsc.parallel_loop(0, 2 * NP, step=16, unroll=4)
    def _(i):
        sl = pl.ds(i, 16)
        Q[sl] = zero16
        O[sl] = zero16
        O2[sl] = zero16

    pltpu.sync_copy(d2_hbm, D2)
    pltpu.sync_copy(zt_hbm.at[pl.ds((7 * H + h0) * NP, NP)], P.at[pl.ds(0, NP)])
    pltpu.sync_copy(zt_hbm.at[pl.ds((7 * H + h0 + 1) * NP, NP)], P.at[pl.ds(NP, NP)])

    def start_chunk(c, buf, sem):
        pltpu.async_copy(packed_hbm.at[pl.ds(c * CH, CH)], buf, sem)

    def wait_chunk(buf, sem):
        pltpu.make_async_copy(packed_hbm.at[pl.ds(0, CH)], buf, sem).wait()

    start_chunk(0, EB0, sem0)

    def process(Pb, buf):
        @plsc.parallel_loop(0, CH, step=16, unroll=8)
        def _(i):
            sd = buf[pl.ds(i, 16)]
            s = lax.shift_right_logical(sd, SB)
            d = sd & msk
            v0 = plsc.load_gather(Pb, [s])
            plsc.addupdate_scatter(O, [d], v0)
            v1 = plsc.load_gather(Pb, [s + nsplat])
            plsc.addupdate_scatter(O2, [d + nsplat], v1)

    def sweep(Pb, Qb, k, fac):
        pltpu.async_copy(
            zt_hbm.at[pl.ds((k * H + h0) * NP, NP)], Z.at[pl.ds(0, NP)], semz)
        pltpu.async_copy(
            zt_hbm.at[pl.ds((k * H + h0 + 1) * NP, NP)], Z.at[pl.ds(NP, NP)], semz)

        def pair(j, _):
            wait_chunk(EB0, sem0)
            start_chunk(2 * j + 1, EB1, sem1)
            process(Pb, EB0)
            wait_chunk(EB1, sem1)
            nxt = lax.rem(2 * j + 2, NCH)
            start_chunk(nxt, EB0, sem0)
            process(Pb, EB1)
            return 0

        lax.fori_loop(0, NCH // 2, pair, 0)

        pltpu.make_async_copy(
            zt_hbm.at[pl.ds(0, NP)], Z.at[pl.ds(0, NP)], semz).wait()
        pltpu.make_async_copy(
            zt_hbm.at[pl.ds(0, NP)], Z.at[pl.ds(NP, NP)], semz).wait()

        @plsc.parallel_loop(0, NP, step=16, unroll=4)
        def _(i):
            sl = pl.ds(i, 16)
            d2 = D2[sl]
            Qb[sl] = Z[sl] - fac * d2 * (O[sl] + O2[sl]) - Qb[sl]
            O[sl] = zero16
            O2[sl] = zero16
            sl1 = pl.ds(NP + i, 16)
            Qb[sl1] = Z[sl1] - fac * d2 * (O[sl1] + O2[sl1]) - Qb[sl1]
            O[sl1] = zero16
            O2[sl1] = zero16

    bufs = [P, Q]
    for k in range(6, 0, -1):
        sweep(bufs[0], bufs[1], k, 2.0)
        bufs = [bufs[1], bufs[0]]
    sweep(bufs[0], bufs[1], 0, 1.0)
    S = bufs[1]

    wait_chunk(EB0, sem0)

    pltpu.sync_copy(S.at[pl.ds(0, NP)], st_hbm.at[pl.ds(h0 * NP, NP)])
    pltpu.sync_copy(S.at[pl.ds(NP, NP)], st_hbm.at[pl.ds((h0 + 1) * NP, NP)])


def _elu_body(st_ref, dsq_ref, x_ref, wc_ref, b_ref, o_ref):
    t = jnp.transpose(st_ref[...])
    t = lax.slice(t, (0, 0), (N, H))
    dsq = jnp.transpose(dsq_ref[...])
    dsq = lax.slice(dsq, (0, 0), (N, 1))
    corr = jnp.dot(x_ref[...], wc_ref[...],
                   preferred_element_type=jnp.float32)
    t = jnp.where(dsq > 0.0, dsq * t, corr) + b_ref[...]
    o_ref[...] = jnp.where(t > 0.0, t, jnp.exp(t) - 1.0)


def _elu(st2, dsq, x, wc, b2):
    return pl.pallas_call(
        _elu_body,
        out_shape=jax.ShapeDtypeStruct((N, H), jnp.float32),
    )(st2, dsq, x, wc, b2)


@jax.jit
def kernel(x, edge_index, W, b):
    src = edge_index[0]
    dst = edge_index[1]

    degp = _deg_kernel(src)

    xp = jnp.pad(x, ((0, NP - N), (0, 0)))
    wf = jnp.transpose(W, (0, 2, 1)).reshape(K * H, D)
    zt, d2, dsq = _dense(xp, wf, degp.reshape(NW, NP))

    packed = src * (1 << SB) + dst

    st = _clenshaw_kernel(packed, zt.reshape(K * H * NP), d2.reshape(NP))

    wc = W[0] - W[2] + W[4] - W[6]
    return _elu(st.reshape(H, NP), dsq, x, wc, b.reshape(1, H))


# speedup vs baseline: 1.0264x; 1.0264x over previous
import functools

import jax
import jax.numpy as jnp
from jax import lax
from jax.experimental import pallas as pl
from jax.experimental.pallas import tpu as pltpu
from jax.experimental.pallas import tpu_sc as plsc

N = 10000
NP = 10240
E = 320000
D = 128
H = 64
K = 8

NC = 2
NS = 16
NW = NC * NS
EPW = E // NW
CH = 16000
NCH = E // CH
FPW = H // NW
SB = 14

_mesh = plsc.VectorSubcoreMesh(core_axis_name="c", subcore_axis_name="s")
_sc_params = pltpu.CompilerParams(needs_layout_passes=False)


def _wid():
    return lax.axis_index("s") * NC + lax.axis_index("c")


@functools.partial(
    pl.kernel,
    out_type=jax.ShapeDtypeStruct((NW * NP,), jnp.float32),
    mesh=_mesh,
    compiler_params=_sc_params,
    scratch_types=[
        pltpu.VMEM((EPW,), jnp.int32),
        pltpu.VMEM((NP,), jnp.float32),
    ],
)
def _deg_kernel(src_hbm, out_hbm, src_v, deg_v):
    wid = _wid()
    zero16 = jnp.zeros((16,), jnp.float32)

    @plsc.parallel_loop(0, NP, step=16, unroll=4)
    def _(i):
        deg_v[pl.ds(i, 16)] = zero16

    pltpu.sync_copy(src_hbm.at[pl.ds(wid * EPW, EPW)], src_v)
    one16 = jnp.ones((16,), jnp.float32)

    @plsc.parallel_loop(0, EPW, step=16, unroll=8)
    def _(i):
        s = src_v[pl.ds(i, 16)]
        plsc.addupdate_scatter(deg_v, [s], one16)

    pltpu.sync_copy(deg_v, out_hbm.at[pl.ds(wid * NP, NP)])


def _dense_body(x_ref, wf_ref, degp_ref, zt_ref, d2_ref, dsq_ref):
    deg = jnp.sum(degp_ref[...], axis=0, keepdims=True)
    dinv = jnp.where(deg > 0.0, lax.rsqrt(deg), 0.0)
    d2_ref[...] = dinv * dinv
    dsq_ref[...] = jnp.where(deg > 0.0, jnp.sqrt(deg), 0.0)
    zt = lax.dot_general(
        wf_ref[...], x_ref[...], (((1,), (1,)), ((), ())),
        preferred_element_type=jnp.float32)
    zt_ref[...] = zt * dinv


def _dense(xp, wf, degp):
    return pl.pallas_call(
        _dense_body,
        out_shape=(
            jax.ShapeDtypeStruct((K * H, NP), jnp.float32),
            jax.ShapeDtypeStruct((1, NP), jnp.float32),
            jax.ShapeDtypeStruct((1, NP), jnp.float32),
        ),
    )(xp, wf, degp)


@functools.partial(
    pl.kernel,
    out_type=jax.ShapeDtypeStruct((H * NP,), jnp.float32),
    mesh=_mesh,
    compiler_params=_sc_params,
    scratch_types=[
        pltpu.VMEM((2 * NP,), jnp.float32),
        pltpu.VMEM((2 * NP,), jnp.float32),
        pltpu.VMEM((2 * NP,), jnp.float32),
        pltpu.VMEM((2 * NP,), jnp.float32),
        pltpu.VMEM((NP,), jnp.float32),
        pltpu.VMEM((CH,), jnp.int32),
        pltpu.VMEM((CH,), jnp.int32),
        pltpu.SemaphoreType.DMA,
        pltpu.SemaphoreType.DMA,
        pltpu.SemaphoreType.DMA,
    ],
)
def _clenshaw_kernel(packed_hbm, zt_hbm, d2_hbm, st_hbm, P, Q, O, Z, D2,
                     EB0, EB1, sem0, sem1, semz):
    wid = _wid()
    h0 = wid * FPW
    zero16 = jnp.zeros((16,), jnp.float32)
    nsplat = jnp.full((16,), NP, jnp.int32)
    msk = jnp.full((16,), (1 << SB) - 1, jnp.int32)

    @plsc.parallel_loop(0, 2 * NP, step=16, unroll=4)
    def _(i):
        sl = pl.ds(i, 16)
        Q[sl] = zero16
        O[sl] = zero16

    pltpu.sync_copy(d2_hbm, D2)
    pltpu.sync_copy(zt_hbm.at[pl.ds((7 * H + h0) * NP, NP)], P.at[pl.ds(0, NP)])
    pltpu.sync_copy(zt_hbm.at[pl.ds((7 * H + h0 + 1) * NP, NP)], P.at[pl.ds(NP, NP)])

    def start_chunk(c, buf, sem):
        pltpu.async_copy(packed_hbm.at[pl.ds(c * CH, CH)], buf, sem)

    def wait_chunk(buf, sem):
        pltpu.make_async_copy(packed_hbm.at[pl.ds(0, CH)], buf, sem).wait()

    start_chunk(0, EB0, sem0)

    def process(Pb, buf):
        @plsc.parallel_loop(0, CH, step=16, unroll=8)
        def _(i):
            sd = buf[pl.ds(i, 16)]
            s = lax.shift_right_logical(sd, SB)
            d = sd & msk
            v0 = plsc.load_gather(Pb, [s])
            plsc.addupdate_scatter(O, [d], v0)
            v1 = plsc.load_gather(Pb, [s + nsplat])
            plsc.addupdate_scatter(O, [d + nsplat], v1)

    def sweep(Pb, Qb, k, fac):
        pltpu.async_copy(
            zt_hbm.at[pl.ds((k * H + h0) * NP, NP)], Z.at[pl.ds(0, NP)], semz)
        pltpu.async_copy(
            zt_hbm.at[pl.ds((k * H + h0 + 1) * NP, NP)], Z.at[pl.ds(NP, NP)], semz)

        def pair(j, _):
            wait_chunk(EB0, sem0)
            start_chunk(2 * j + 1, EB1, sem1)
            process(Pb, EB0)
            wait_chunk(EB1, sem1)
            nxt = lax.rem(2 * j + 2, NCH)
            start_chunk(nxt, EB0, sem0)
            process(Pb, EB1)
            return 0

        lax.fori_loop(0, NCH // 2, pair, 0)

        pltpu.make_async_copy(
            zt_hbm.at[pl.ds(0, NP)], Z.at[pl.ds(0, NP)], semz).wait()
        pltpu.make_async_copy(
            zt_hbm.at[pl.ds(0, NP)], Z.at[pl.ds(NP, NP)], semz).wait()

        @plsc.parallel_loop(0, NP, step=16, unroll=4)
        def _(i):
            sl = pl.ds(i, 16)
            d2 = D2[sl]
            Qb[sl] = Z[sl] - fac * d2 * O[sl] - Qb[sl]
            O[sl] = zero16
            sl1 = pl.ds(NP + i, 16)
            Qb[sl1] = Z[sl1] - fac * d2 * O[sl1] - Qb[sl1]
            O[sl1] = zero16

    bufs = [P, Q]
    for k in range(6, 0, -1):
        sweep(bufs[0], bufs[1], k, 2.0)
        bufs = [bufs[1], bufs[0]]
    sweep(bufs[0], bufs[1], 0, 1.0)
    S = bufs[1]

    wait_chunk(EB0, sem0)

    pltpu.sync_copy(S.at[pl.ds(0, NP)], st_hbm.at[pl.ds(h0 * NP, NP)])
    pltpu.sync_copy(S.at[pl.ds(NP, NP)], st_hbm.at[pl.ds((h0 + 1) * NP, NP)])


def _elu_body(st_ref, dsq_ref, x_ref, wc_ref, b_ref, o_ref):
    t = jnp.transpose(st_ref[...])
    t = lax.slice(t, (0, 0), (N, H))
    dsq = jnp.transpose(dsq_ref[...])
    dsq = lax.slice(dsq, (0, 0), (N, 1))
    corr = jnp.dot(x_ref[...], wc_ref[...],
                   preferred_element_type=jnp.float32)
    t = jnp.where(dsq > 0.0, dsq * t, corr) + b_ref[...]
    o_ref[...] = jnp.where(t > 0.0, t, jnp.exp(t) - 1.0)


def _elu(st2, dsq, x, wc, b2):
    return pl.pallas_call(
        _elu_body,
        out_shape=jax.ShapeDtypeStruct((N, H), jnp.float32),
    )(st2, dsq, x, wc, b2)


@jax.jit
def kernel(x, edge_index, W, b):
    src = edge_index[0]
    dst = edge_index[1]

    degp = _deg_kernel(src)

    xp = jnp.pad(x, ((0, NP - N), (0, 0)))
    wf = jnp.transpose(W, (0, 2, 1)).reshape(K * H, D)
    zt, d2, dsq = _dense(xp, wf, degp.reshape(NW, NP))

    packed = src * (1 << SB) + dst

    st = _clenshaw_kernel(packed, zt.reshape(K * H * NP), d2.reshape(NP))

    wc = W[0] - W[2] + W[4] - W[6]
    return _elu(st.reshape(H, NP), dsq, x, wc, b.reshape(1, H))
